# Initial kernel scaffold; baseline (speedup 1.0000x reference)
import jax
import jax.numpy as jnp
from jax.experimental import pallas as pl


def kernel(loc_p, loc_t, conf_p, conf_t):
    raise NotImplementedError("write your pallas kernel here")


# speedup vs baseline: 8.5713x; 8.5713x over previous
import functools

import jax
import jax.numpy as jnp
from jax.experimental import pallas as pl
from jax.experimental.pallas import tpu as pltpu

NUM_CLASSES = 21


def _stage1_body(conf_ref, t_ref, locp_ref, loct_ref, csel_ref, stat_ref,
                 npos_ref):
    conf = conf_ref[0]
    t = t_ref[0]
    pos = t > 0

    m = jnp.max(conf, axis=0, keepdims=True)
    e = jnp.exp(conf - m)
    s = jnp.sum(e, axis=0, keepdims=True)
    lse = jnp.log(s) + m
    cls = jax.lax.broadcasted_iota(jnp.int32, conf.shape, 0)
    tgt = jnp.sum(jnp.where(cls == t, conf, 0.0), axis=0, keepdims=True)
    ce = lse - tgt

    csel_ref[0] = jnp.maximum(jnp.where(pos, 0.0, ce), 0.0)

    pos_ce = jnp.sum(jnp.where(pos, ce, 0.0))

    d = locp_ref[0] - loct_ref[0]
    ad = jnp.abs(d)
    sl1 = jnp.where(ad < 1.0, 0.5 * d * d, ad - 0.5)
    loc = jnp.sum(sl1 * pos.astype(jnp.float32))

    stat_ref[...] = (pos_ce + loc).reshape(1, 1, 1)
    npos_ref[...] = jnp.sum(pos.astype(jnp.int32)).reshape(1, 1, 1)


def _stage2_body(csel_ref, stat_ref, npos_ref, out_ref, *, n):
    b = csel_ref.shape[0]
    x = csel_ref[...]
    keys = jax.lax.bitcast_convert_type(x, jnp.int32)
    npos = npos_ref[...]
    k = jnp.minimum(3 * npos, n - npos)

    def body(_, carry):
        lo, hi = carry
        mid = lo + ((hi - lo) >> 1)
        cnt = jnp.sum((keys > mid).astype(jnp.int32), axis=1, keepdims=True)
        take = cnt < k
        return jnp.where(take, lo, mid + 1), jnp.where(take, mid, hi)

    lo0 = jnp.zeros((b, 1), jnp.int32)
    hi0 = jnp.full((b, 1), 0x7F800000, jnp.int32)
    v, _ = jax.lax.fori_loop(0, 31, body, (lo0, hi0))

    gt = keys > v
    cnt_gt = jnp.sum(gt.astype(jnp.int32), axis=1, keepdims=True)
    sum_gt = jnp.sum(jnp.where(gt, x, 0.0), axis=1, keepdims=True)
    vval = jax.lax.bitcast_convert_type(v, jnp.float32)
    topk = sum_gt + (k - cnt_gt).astype(jnp.float32) * vval

    num = jnp.sum(stat_ref[...]) + jnp.sum(topk)
    den = jnp.sum(npos).astype(jnp.float32)
    out_ref[...] = (num / den).reshape(1, 1)


def kernel(loc_p, loc_t, conf_p, conf_t):
    b, n, _ = loc_p.shape
    conf_tr = jnp.transpose(conf_p, (0, 2, 1))
    locp_tr = jnp.transpose(loc_p, (0, 2, 1))
    loct_tr = jnp.transpose(loc_t, (0, 2, 1))
    t3 = conf_t.astype(jnp.int32).reshape(b, 1, n)

    csel, stat, npos = pl.pallas_call(
        _stage1_body,
        grid=(b,),
        in_specs=[
            pl.BlockSpec((1, NUM_CLASSES, n), lambda i: (i, 0, 0)),
            pl.BlockSpec((1, 1, n), lambda i: (i, 0, 0)),
            pl.BlockSpec((1, 4, n), lambda i: (i, 0, 0)),
            pl.BlockSpec((1, 4, n), lambda i: (i, 0, 0)),
        ],
        out_specs=[
            pl.BlockSpec((1, 1, n), lambda i: (i, 0, 0)),
            pl.BlockSpec((1, 1, 1), lambda i: (i, 0, 0)),
            pl.BlockSpec((1, 1, 1), lambda i: (i, 0, 0)),
        ],
        out_shape=[
            jax.ShapeDtypeStruct((b, 1, n), jnp.float32),
            jax.ShapeDtypeStruct((b, 1, 1), jnp.float32),
            jax.ShapeDtypeStruct((b, 1, 1), jnp.int32),
        ],
    )(conf_tr, t3, locp_tr, loct_tr)

    out = pl.pallas_call(
        functools.partial(_stage2_body, n=n),
        in_specs=[
            pl.BlockSpec((b, n), lambda: (0, 0)),
            pl.BlockSpec((b, 1), lambda: (0, 0)),
            pl.BlockSpec((b, 1), lambda: (0, 0)),
        ],
        out_specs=pl.BlockSpec((1, 1), lambda: (0, 0)),
        out_shape=jax.ShapeDtypeStruct((1, 1), jnp.float32),
    )(csel.reshape(b, n), stat.reshape(b, 1), npos.reshape(b, 1))

    return out[0, 0]
